# Initial kernel scaffold; baseline (speedup 1.0000x reference)
import jax
import jax.numpy as jnp
from jax.experimental import pallas as pl


def kernel(node_emb, start_pos, params, time_table):
    raise NotImplementedError("write your pallas kernel here")


# speedup vs baseline: 3.8987x; 3.8987x over previous
import functools

import jax
import jax.numpy as jnp
from jax import lax
from jax.experimental import pallas as pl
from jax.experimental.pallas import tpu as pltpu
from jax.experimental.pallas import tpu_sc as plsc

D = 128
A = 16
STEPS = 4
B = 8
N = 10000
NBK = 512
NB = 20
NPAD = NB * NBK
SA = STEPS * A
EPS = 1e-5
NEG = -1e30


def _lrelu(x):
    return jnp.where(x > 0, x, 0.01 * x)


def _ln(x, g, b):
    m = jnp.mean(x, -1, keepdims=True)
    v = jnp.mean(x * x, -1, keepdims=True) - m * m
    return (x - m) * lax.rsqrt(v + EPS) * g + b


def _mm(x, w):
    return lax.dot_general(x, w, (((x.ndim - 1,), (1,)), ((), ())),
                           preferred_element_type=jnp.float32)


def _bmm(x, y, cdim):
    return lax.dot_general(x, y, (((2,), (cdim,)), ((0,), (0,))),
                           preferred_element_type=jnp.float32)


def _k1_body(ne_ref, k1g_ref, k1b_ref, k1W_ref, k1bb_ref, oW_ref, ob_ref,
             k_ref, out_ref):
    e = ne_ref[...]
    m = jnp.mean(e, -1, keepdims=True)
    v = jnp.mean(e * e, -1, keepdims=True) - m * m
    nrm = (e - m) * lax.rsqrt(v + EPS)
    gA = k1g_ref[:, :D]
    gB = k1g_ref[:, D:]
    bA = k1b_ref[:, :D]
    bB = k1b_ref[:, D:]
    WA = k1W_ref[:, :D]
    WB = k1W_ref[:, D:]
    Wc = WA * gA + WB * gB
    bc = _mm(bA, WA) + _mm(bB, WB) + k1bb_ref[...]
    nf = nrm.reshape(B * NBK, D)
    k_ref[...] = (_mm(nf, Wc) + bc).reshape(B, NBK, D)
    ef = e.reshape(B * NBK, D)
    out_ref[...] = (_mm(ef, oW_ref[...]) + ob_ref[...]).reshape(B, NBK, D)


def _run_k1(node_emb, p1):
    f32 = jnp.float32
    return pl.pallas_call(
        _k1_body,
        grid=(NB,),
        in_specs=[
            pl.BlockSpec((B, NBK, D), lambda j: (0, j, 0)),
            pl.BlockSpec((1, 2 * D), lambda j: (0, 0)),
            pl.BlockSpec((1, 2 * D), lambda j: (0, 0)),
            pl.BlockSpec((D, 2 * D), lambda j: (0, 0)),
            pl.BlockSpec((1, D), lambda j: (0, 0)),
            pl.BlockSpec((D, D), lambda j: (0, 0)),
            pl.BlockSpec((1, D), lambda j: (0, 0)),
        ],
        out_specs=[
            pl.BlockSpec((B, NBK, D), lambda j: (0, j, 0)),
            pl.BlockSpec((B, NBK, D), lambda j: (0, j, 0)),
        ],
        out_shape=[
            jax.ShapeDtypeStruct((B, NPAD, D), f32),
            jax.ShapeDtypeStruct((B, N, D), f32),
        ],
    )(node_emb, p1['k1_g'], p1['k1_b'], p1['k1_W'], p1['k1_bb'],
      p1['out_W'], p1['out_b'])


def _make_sc_gather():
    info = plsc.get_sparse_core_info()
    NC = info.num_cores
    mesh = plsc.VectorSubcoreMesh(core_axis_name="c", subcore_axis_name="s")
    n_workers = 16
    rpw = (B * A) // n_workers

    @functools.partial(
        pl.kernel,
        out_type=jax.ShapeDtypeStruct((B * A, D), jnp.float32),
        mesh=mesh,
        scratch_types=[
            pltpu.VMEM((rpw,), jnp.int32),
            pltpu.VMEM((rpw, D), jnp.float32),
            pltpu.SemaphoreType.DMA,
        ],
    )
    def gather_rows(tbl_hbm, idx_hbm, out_hbm, idx_v, rows_v, sem):
        wid = lax.axis_index("s") * NC + lax.axis_index("c")

        @pl.when(wid < n_workers)
        def _():
            base = wid * rpw
            pltpu.sync_copy(idx_hbm.at[pl.ds(base, rpw)], idx_v)
            pltpu.async_copy(tbl_hbm.at[idx_v], rows_v, sem).wait()
            pltpu.sync_copy(rows_v, out_hbm.at[pl.ds(base, rpw)])

    return gather_rows


def _agent_body(t, k_any, kidx_ref, a_ref, pos_ref, cur0_ref, upos_ref,
                uval_ref, tt_ref,
                teW1, teb1, teW2, teb2, agtW, agtb, ndtW, ndtb,
                gpg, gpb, gpW1, gpb1, gpW2, gpb2,
                agg_, agb_, agW1, agb1, agW2, agb2,
                msgg, msgb, msgW, msgbb,
                ndg, ndb, ndW1, ndb1, ndW2, ndb2,
                qg, qb, qW, qbb, k1g, k1b, k1W, k1bb,
                k_any_out, a_out, q_out, upos_out, uval_out,
                knew_v, sem):
    pos = pos_ref[...]
    a_emb = a_ref[...]
    cur0 = cur0_ref[...].reshape(B, A, D)

    tt = tt_ref[...]
    t1 = _lrelu(_mm(tt, teW1[...]) + teb1[...])
    t2 = _mm(t1, teW2[...]) + teb2[...]
    lt2 = _lrelu(t2)
    ag_add = _mm(lt2, agtW[...]) + agtb[...]
    nd_add = _mm(lt2, ndtW[...]) + ndtb[...]

    pos3 = pos[:, :, None]
    cur = cur0
    for j in range(t * A):
        s, a2 = j // A, j % A
        hv = upos_ref[s, :, a2:a2 + 1][:, :, None]
        cur = jnp.where(pos3 == hv, uval_ref[s, :, a2:a2 + 1, :], cur)

    am = jnp.mean(a_emb, axis=1)
    gv = _ln(am, gpg[...], gpb[...])
    gv = _lrelu(_mm(gv, gpW1[...]) + gpb1[...])
    gvec = _mm(gv, gpW2[...]) + gpb2[...]
    gvec_b = jnp.broadcast_to(gvec[:, None, :], (B, A, D))

    ag_in = jnp.concatenate([a_emb, cur, gvec_b], -1) + ag_add[0]
    h = _ln(ag_in, agg_[...], agb_[...]).reshape(B * A, 3 * D)
    h = _lrelu(_mm(h, agW1[...]) + agb1[...])
    h = _mm(h, agW2[...]) + agb2[...]
    val, gate = h[:, :D], h[:, D:]
    g = jax.nn.sigmoid(gate)
    a_new = g * a_emb.reshape(B * A, D) + (1 - g) * jnp.tanh(val)
    a_new3 = a_new.reshape(B, A, D)

    msg = _ln(a_new3, msgg[...], msgb[...])
    msg = jax.nn.relu(_mm(msg.reshape(B * A, D), msgW[...]) + msgbb[...])
    msg3 = msg.reshape(B, A, D)
    eqm = (pos[:, :, None] == pos[:, None, :]).astype(jnp.float32)
    agg_cur = _bmm(eqm, msg3, 1)

    nd_in = jnp.concatenate([cur, agg_cur, gvec_b, cur0], -1) + nd_add[0]
    h = _ln(nd_in, ndg[...], ndb[...]).reshape(B * A, 4 * D)
    h = _lrelu(_mm(h, ndW1[...]) + ndb1[...])
    h = _mm(h, ndW2[...]) + ndb2[...]
    val, gate = h[:, :D], h[:, D:]
    g = jax.nn.sigmoid(gate)
    new_node = g * cur.reshape(B * A, D) + (1 - g) * jnp.tanh(val)
    nn3 = new_node.reshape(B, A, D)

    upos_out[...] = upos_ref[...]
    uval_out[...] = uval_ref[...]
    upos_out[t] = pos
    uval_out[t] = nn3

    kin = jnp.concatenate([nn3, cur0], -1)
    kn = _ln(kin, k1g[...], k1b[...])
    k_new = (_mm(kn.reshape(B * A, 2 * D), k1W[...]) + k1bb[...]
             ).reshape(B, A, D)
    kw = k_new
    for a2 in range(A):
        eq3 = pos3 == pos3[:, a2:a2 + 1, :]
        kw = jnp.where(eq3, k_new[:, a2:a2 + 1, :], kw)

    q = _ln(a_new, qg[...], qb[...])
    q_out[...] = (_mm(q, qW[...]) + qbb[...]).reshape(B, A, D)
    a_out[...] = a_new3

    knew_v[...] = kw.reshape(B * A, D)

    def _start(i, _):
        pltpu.make_async_copy(
            knew_v.at[pl.ds(i, 1), :],
            k_any_out.at[pl.ds(kidx_ref[i], 1), :],
            sem).start()
        return 0

    lax.fori_loop(0, B * A, _start, 0)

    def _wait(i, _):
        pltpu.make_async_copy(
            knew_v.at[pl.ds(0, 1), :],
            k_any_out.at[pl.ds(0, 1), :],
            sem).wait()
        return 0

    lax.fori_loop(0, B * A, _wait, 0)


def _run_agent(t, k_flat, kidx, a_emb, pos, cur0, upos, uval, tt_row, p1):
    f32 = jnp.float32
    i32 = jnp.int32
    full = pl.BlockSpec(memory_space=pl.ANY)
    vm = pl.BlockSpec(memory_space=pltpu.MemorySpace.VMEM)
    sm = pl.BlockSpec(memory_space=pltpu.MemorySpace.SMEM)
    w = p1
    return pl.pallas_call(
        functools.partial(_agent_body, t),
        in_specs=[full, sm] + [vm] * 44,
        out_specs=[full, vm, vm, vm, vm],
        out_shape=[
            jax.ShapeDtypeStruct((B * NPAD, D), f32),
            jax.ShapeDtypeStruct((B, A, D), f32),
            jax.ShapeDtypeStruct((B, A, D), f32),
            jax.ShapeDtypeStruct((STEPS, B, A), i32),
            jax.ShapeDtypeStruct((STEPS, B, A, D), f32),
        ],
        input_output_aliases={0: 0},
        scratch_shapes=[
            pltpu.VMEM((B * A, D), f32),
            pltpu.SemaphoreType.DMA,
        ],
    )(k_flat, kidx, a_emb, pos, cur0, upos, uval, tt_row,
      w['te_W1'], w['te_b1'], w['te_W2'], w['te_b2'],
      w['ag_tW'], w['ag_tb'], w['nd_tW'], w['nd_tb'],
      w['gp_g'], w['gp_b'], w['gp_W1'], w['gp_b1'], w['gp_W2'], w['gp_b2'],
      w['ag_g'], w['ag_b'], w['ag_W1'], w['ag_b1'], w['ag_W2'], w['ag_b2'],
      w['msg_g'], w['msg_b'], w['msg_W'], w['msg_bb'],
      w['nd_g'], w['nd_b'], w['nd_W1'], w['nd_b1'], w['nd_W2'], w['nd_b2'],
      w['q_g'], w['q_b'], w['q_W'], w['q_bb'],
      w['k1_g'], w['k1_b'], w['k1_W'], w['k1_bb'])


def _argmax_body(k_ref, q_ref, pos_ref, vis_ref, cu_ref,
                 pos_out, vis_out, bv_ref, bi_ref):
    j = pl.program_id(0)
    kblk = k_ref[...]
    q = q_ref[...]
    pos = pos_ref[...]

    scores = _bmm(q, kblk, 2) * (1.0 / (D ** 0.5))

    n_g = j * NBK + lax.broadcasted_iota(jnp.int32, (1, NBK), 1)
    member = pos[:, 0][:, None] == n_g
    for a2 in range(1, A):
        member = member | (pos[:, a2][:, None] == n_g)
    vis = jnp.where(member, 1.0, vis_ref[...])
    cu = cu_ref[0]
    scores = scores + cu * vis[:, None, :]
    valid = n_g < N
    scores = jnp.where(valid[:, None, :], scores, NEG)
    vis_out[...] = vis * 0.9

    sf = scores.reshape(B * A, NBK)
    bm = jnp.max(sf, axis=1, keepdims=True)
    iot = lax.broadcasted_iota(jnp.int32, (B * A, NBK), 1) + j * NBK
    idx = jnp.min(jnp.where(sf == bm, iot, jnp.int32(2 ** 30)),
                  axis=1, keepdims=True)

    @pl.when(j == 0)
    def _():
        bv_ref[...] = bm
        bi_ref[...] = idx

    @pl.when(j > 0)
    def _():
        upd = bm > bv_ref[...]
        bv_ref[...] = jnp.where(upd, bm, bv_ref[...])
        bi_ref[...] = jnp.where(upd, idx, bi_ref[...])

    @pl.when(j == NB - 1)
    def _():
        pos_out[...] = bi_ref[...].reshape(B, A)


def _run_argmax(k3, q, pos, visited, cu):
    f32 = jnp.float32
    i32 = jnp.int32
    return pl.pallas_call(
        _argmax_body,
        grid=(NB,),
        in_specs=[
            pl.BlockSpec((B, NBK, D), lambda j: (0, j, 0)),
            pl.BlockSpec((B, A, D), lambda j: (0, 0, 0)),
            pl.BlockSpec((B, A), lambda j: (0, 0)),
            pl.BlockSpec((B, NBK), lambda j: (0, j)),
            pl.BlockSpec(memory_space=pltpu.MemorySpace.SMEM),
        ],
        out_specs=[
            pl.BlockSpec((B, A), lambda j: (0, 0)),
            pl.BlockSpec((B, NBK), lambda j: (0, j)),
        ],
        out_shape=[
            jax.ShapeDtypeStruct((B, A), i32),
            jax.ShapeDtypeStruct((B, NPAD), f32),
        ],
        scratch_shapes=[
            pltpu.VMEM((B * A, 1), f32),
            pltpu.VMEM((B * A, 1), i32),
        ],
    )(k3, q, pos, visited, cu)


def _fix_body(out_any, fidx_ref, upos_ref, uval_ref, oW_ref, ob_ref,
              out_any_out, rows_v, sem):
    uval = uval_ref[...]
    rows = (_mm(uval.reshape(SA * B, D), oW_ref[...]) + ob_ref[...]
            ).reshape(STEPS, B, A, D)
    rw = rows
    upos4 = upos_ref[...][..., None]
    for jj in range(SA):
        s2, a2 = jj // A, jj % A
        hv = upos4[s2:s2 + 1, :, a2:a2 + 1, :]
        rw = jnp.where(upos4 == hv, rows[s2:s2 + 1, :, a2:a2 + 1, :], rw)
    rows_v[...] = rw.reshape(SA * B, D)

    def _start(i, _):
        pltpu.make_async_copy(
            rows_v.at[pl.ds(i, 1), :],
            out_any_out.at[pl.ds(fidx_ref[i], 1), :],
            sem).start()
        return 0

    lax.fori_loop(0, SA * B, _start, 0)

    def _wait(i, _):
        pltpu.make_async_copy(
            rows_v.at[pl.ds(0, 1), :],
            out_any_out.at[pl.ds(0, 1), :],
            sem).wait()
        return 0

    lax.fori_loop(0, SA * B, _wait, 0)


def _run_fix(out_flat, fidx, upos, uval, p1):
    f32 = jnp.float32
    full = pl.BlockSpec(memory_space=pl.ANY)
    vm = pl.BlockSpec(memory_space=pltpu.MemorySpace.VMEM)
    sm = pl.BlockSpec(memory_space=pltpu.MemorySpace.SMEM)
    return pl.pallas_call(
        _fix_body,
        in_specs=[full, sm, vm, vm, vm, vm],
        out_specs=[full],
        out_shape=[jax.ShapeDtypeStruct((B * N, D), f32)],
        input_output_aliases={0: 0},
        scratch_shapes=[
            pltpu.VMEM((SA * B, D), f32),
            pltpu.SemaphoreType.DMA,
        ],
    )(out_flat, fidx, upos, uval, p1['out_W'], p1['out_b'])[0]


def kernel(node_emb, start_pos, params, time_table):
    p = params
    f32 = jnp.float32
    i32 = jnp.int32

    def row(v):
        return v.reshape(1, -1).astype(f32)

    p1 = {k: row(p[k]) for k in (
        'te_b1', 'te_b2', 'ag_tb', 'nd_tb', 'gp_g', 'gp_b', 'gp_b1',
        'gp_b2', 'ag_g', 'ag_b', 'ag_b1', 'ag_b2', 'msg_g', 'msg_b',
        'msg_bb', 'nd_g', 'nd_b', 'nd_b1', 'nd_b2', 'q_g', 'q_b', 'q_bb',
        'k1_g', 'k1_b', 'k1_bb', 'out_b')}
    for k in ('te_W1', 'te_W2', 'ag_tW', 'nd_tW', 'gp_W1', 'gp_W2',
              'ag_W1', 'ag_W2', 'msg_W', 'nd_W1', 'nd_W2', 'q_W', 'k1_W',
              'out_W'):
        p1[k] = p[k].astype(f32)

    k3, out_base = _run_k1(node_emb, p1)
    k_flat = k3.reshape(B * NPAD, D)
    out_flat = out_base.reshape(B * N, D)

    sc_gather = _make_sc_gather()
    ne_flat = node_emb.reshape(B * N, D)

    b_off_N = (jnp.arange(B, dtype=i32) * N)[:, None]
    b_off_P = (jnp.arange(B, dtype=i32) * NPAD)[:, None]
    cu = (p['explored'] - p['unexplored']).reshape(1).astype(f32)

    a_emb = jnp.broadcast_to(p['agent_emb'][None].astype(f32), (B, A, D))
    pos = jnp.broadcast_to(start_pos[:, None], (B, A)).astype(i32)
    upos = jnp.full((STEPS, B, A), -1, i32)
    uval = jnp.zeros((STEPS, B, A, D), f32)
    visited = jnp.zeros((B, NPAD), f32)

    for t in range(STEPS):
        gidx = (b_off_N + pos).reshape(B * A)
        kidx = (b_off_P + pos).reshape(B * A)
        cur0 = sc_gather(ne_flat, gidx)
        tt_row = lax.dynamic_slice(time_table, (t, 0), (1, D))
        k_flat, a_emb, q, upos, uval = _run_agent(
            t, k_flat, kidx, a_emb, pos, cur0, upos, uval, tt_row, p1)
        k3v = k_flat.reshape(B, NPAD, D)
        pos, visited = _run_argmax(k3v, q, pos, visited, cu)

    fidx = (b_off_N[None] + upos).transpose(0, 1, 2).reshape(SA * B)
    out_flat = _run_fix(out_flat, fidx, upos, uval, p1)
    return out_flat.reshape(B, N, D)


# speedup vs baseline: 3.9473x; 1.0125x over previous
import functools

import jax
import jax.numpy as jnp
from jax import lax
from jax.experimental import pallas as pl
from jax.experimental.pallas import tpu as pltpu
from jax.experimental.pallas import tpu_sc as plsc

D = 128
A = 16
STEPS = 4
B = 8
N = 10000
NBK = 512
NB = 20
NPAD = NB * NBK
SA = STEPS * A
EPS = 1e-5
NEG = -1e30

_SEG = [
    ('te_b1', 256), ('te_b2', 256), ('ag_tb', 384), ('nd_tb', 512),
    ('gp_g', 128), ('gp_b', 128), ('gp_b1', 256), ('gp_b2', 128),
    ('ag_g', 384), ('ag_b', 384), ('ag_b1', 512), ('ag_b2', 256),
    ('msg_g', 128), ('msg_b', 128), ('msg_bb', 128),
    ('nd_g', 512), ('nd_b', 512), ('nd_b1', 512), ('nd_b2', 256),
    ('q_g', 128), ('q_b', 128), ('q_bb', 128),
    ('k1_g', 256), ('k1_b', 256), ('k1_bb', 128), ('out_b', 128),
]
_POFF = {}
_off = 0
for _n, _l in _SEG:
    _POFF[_n] = (_off, _l)
    _off += _l
PK = _off


def _pv(ref, name):
    o, n = _POFF[name]
    return ref[:, o:o + n]


def _lrelu(x):
    return jnp.where(x > 0, x, 0.01 * x)


def _ln(x, g, b):
    m = jnp.mean(x, -1, keepdims=True)
    v = jnp.mean(x * x, -1, keepdims=True) - m * m
    return (x - m) * lax.rsqrt(v + EPS) * g + b


def _mm(x, w):
    return lax.dot_general(x, w, (((x.ndim - 1,), (1,)), ((), ())),
                           preferred_element_type=jnp.float32)


def _bmm(x, y, cdim):
    return lax.dot_general(x, y, (((2,), (cdim,)), ((0,), (0,))),
                           preferred_element_type=jnp.float32)


def _k1_body(ne_ref, pvec, k1W_ref, oW_ref, k_ref, out_ref):
    e = ne_ref[...]
    m = jnp.mean(e, -1, keepdims=True)
    v = jnp.mean(e * e, -1, keepdims=True) - m * m
    nrm = (e - m) * lax.rsqrt(v + EPS)
    k1g = _pv(pvec, 'k1_g')
    k1b = _pv(pvec, 'k1_b')
    gA, gB = k1g[:, :D], k1g[:, D:]
    bA, bB = k1b[:, :D], k1b[:, D:]
    WA = k1W_ref[:, :D]
    WB = k1W_ref[:, D:]
    Wc = WA * gA + WB * gB
    bc = _mm(bA, WA) + _mm(bB, WB) + _pv(pvec, 'k1_bb')
    nf = nrm.reshape(B * NBK, D)
    k_ref[...] = (_mm(nf, Wc) + bc).reshape(B, NBK, D)
    ef = e.reshape(B * NBK, D)
    out_ref[...] = (_mm(ef, oW_ref[...]) + _pv(pvec, 'out_b')
                    ).reshape(B, NBK, D)


def _run_k1(node_emb, pvec, p):
    f32 = jnp.float32
    return pl.pallas_call(
        _k1_body,
        grid=(NB,),
        in_specs=[
            pl.BlockSpec((B, NBK, D), lambda j: (0, j, 0)),
            pl.BlockSpec((1, PK), lambda j: (0, 0)),
            pl.BlockSpec((D, 2 * D), lambda j: (0, 0)),
            pl.BlockSpec((D, D), lambda j: (0, 0)),
        ],
        out_specs=[
            pl.BlockSpec((B, NBK, D), lambda j: (0, j, 0)),
            pl.BlockSpec((B, NBK, D), lambda j: (0, j, 0)),
        ],
        out_shape=[
            jax.ShapeDtypeStruct((B, NPAD, D), f32),
            jax.ShapeDtypeStruct((B, N, D), f32),
        ],
    )(node_emb, pvec, p['k1_W'], p['out_W'])


def _make_sc_gather():
    mesh = plsc.VectorSubcoreMesh(core_axis_name="c", subcore_axis_name="s")
    info = plsc.get_sparse_core_info()
    NC = info.num_cores

    @functools.partial(
        pl.kernel,
        out_type=jax.ShapeDtypeStruct((B * A, D), jnp.float32),
        mesh=mesh,
        scratch_types=[
            pltpu.VMEM((A,), jnp.int32),
            pltpu.VMEM((A, D), jnp.float32),
            pltpu.SemaphoreType.DMA,
        ],
    )
    def gather_rows(tbl_hbm, pos_hbm, out_hbm, idx_v, rows_v, sem):
        wid = lax.axis_index("s") * NC + lax.axis_index("c")

        @pl.when(wid < B)
        def _():
            pltpu.sync_copy(pos_hbm.at[wid], idx_v)
            idx_v[...] = idx_v[...] + wid * N
            pltpu.async_copy(tbl_hbm.at[idx_v], rows_v, sem).wait()
            pltpu.sync_copy(rows_v, out_hbm.at[pl.ds(wid * A, A)])

    return gather_rows


def _agent_body(t, k_any, pos_sm, a_ref, pos_ref, cur0_ref, tt_ref, pvec,
                upos_ref, uval_ref,
                teW1, teW2, agtW, ndtW, gpW1, gpW2, agW1, agW2, msgW,
                ndW1, ndW2, qW, k1W,
                k_any_out, a_out, q_out, upos_out, uval_out,
                knew_v, sem):
    pos = pos_ref[...]
    if t == 0:
        a_emb = jnp.broadcast_to(a_ref[...][None], (B, A, D))
    else:
        a_emb = a_ref[...]
    cur0 = cur0_ref[...].reshape(B, A, D)

    tt = tt_ref[t:t + 1, :]
    t1 = _lrelu(_mm(tt, teW1[...]) + _pv(pvec, 'te_b1'))
    t2 = _mm(t1, teW2[...]) + _pv(pvec, 'te_b2')
    lt2 = _lrelu(t2)
    ag_add = _mm(lt2, agtW[...]) + _pv(pvec, 'ag_tb')
    nd_add = _mm(lt2, ndtW[...]) + _pv(pvec, 'nd_tb')

    pos3 = pos[:, :, None]
    cur = cur0
    for j in range(t * A):
        s, a2 = j // A, j % A
        hv = upos_ref[s, :, a2:a2 + 1][:, :, None]
        cur = jnp.where(pos3 == hv, uval_ref[s, :, a2:a2 + 1, :], cur)

    am = jnp.mean(a_emb, axis=1)
    gv = _ln(am, _pv(pvec, 'gp_g'), _pv(pvec, 'gp_b'))
    gv = _lrelu(_mm(gv, gpW1[...]) + _pv(pvec, 'gp_b1'))
    gvec = _mm(gv, gpW2[...]) + _pv(pvec, 'gp_b2')
    gvec_b = jnp.broadcast_to(gvec[:, None, :], (B, A, D))

    ag_in = jnp.concatenate([a_emb, cur, gvec_b], -1) + ag_add[0]
    h = _ln(ag_in, _pv(pvec, 'ag_g'), _pv(pvec, 'ag_b')).reshape(B * A, 3 * D)
    h = _lrelu(_mm(h, agW1[...]) + _pv(pvec, 'ag_b1'))
    h = _mm(h, agW2[...]) + _pv(pvec, 'ag_b2')
    val, gate = h[:, :D], h[:, D:]
    g = jax.nn.sigmoid(gate)
    a_new = g * a_emb.reshape(B * A, D) + (1 - g) * jnp.tanh(val)
    a_new3 = a_new.reshape(B, A, D)

    msg = _ln(a_new3, _pv(pvec, 'msg_g'), _pv(pvec, 'msg_b'))
    msg = jax.nn.relu(_mm(msg.reshape(B * A, D), msgW[...])
                      + _pv(pvec, 'msg_bb'))
    msg3 = msg.reshape(B, A, D)
    eqm = (pos[:, :, None] == pos[:, None, :]).astype(jnp.float32)
    agg_cur = _bmm(eqm, msg3, 1)

    nd_in = jnp.concatenate([cur, agg_cur, gvec_b, cur0], -1) + nd_add[0]
    h = _ln(nd_in, _pv(pvec, 'nd_g'), _pv(pvec, 'nd_b')).reshape(B * A, 4 * D)
    h = _lrelu(_mm(h, ndW1[...]) + _pv(pvec, 'nd_b1'))
    h = _mm(h, ndW2[...]) + _pv(pvec, 'nd_b2')
    val, gate = h[:, :D], h[:, D:]
    g = jax.nn.sigmoid(gate)
    new_node = g * cur.reshape(B * A, D) + (1 - g) * jnp.tanh(val)
    nn3 = new_node.reshape(B, A, D)

    if t == 0:
        upos_out[...] = jnp.full((STEPS, B, A), -1, jnp.int32)
        uval_out[...] = jnp.zeros((STEPS, B, A, D), jnp.float32)
    else:
        upos_out[...] = upos_ref[...]
        uval_out[...] = uval_ref[...]
    upos_out[t] = pos
    uval_out[t] = nn3

    kin = jnp.concatenate([nn3, cur0], -1)
    kn = _ln(kin, _pv(pvec, 'k1_g'), _pv(pvec, 'k1_b'))
    k_new = (_mm(kn.reshape(B * A, 2 * D), k1W[...]) + _pv(pvec, 'k1_bb')
             ).reshape(B, A, D)
    kw = k_new
    for a2 in range(A):
        eq3 = pos3 == pos3[:, a2:a2 + 1, :]
        kw = jnp.where(eq3, k_new[:, a2:a2 + 1, :], kw)

    a_out[...] = a_new3
    if t == STEPS - 1:
        q_out[...] = a_new3
        return

    q = _ln(a_new, _pv(pvec, 'q_g'), _pv(pvec, 'q_b'))
    q_out[...] = (_mm(q, qW[...]) + _pv(pvec, 'q_bb')).reshape(B, A, D)

    knew_v[...] = kw.reshape(B * A, D)

    def _start(i, _):
        b = i // A
        a = i % A
        row = pos_sm[b, a] + b * NPAD
        pltpu.make_async_copy(
            knew_v.at[pl.ds(i, 1), :],
            k_any_out.at[pl.ds(row, 1), :],
            sem).start()
        return 0

    lax.fori_loop(0, B * A, _start, 0)

    def _wait(i, _):
        pltpu.make_async_copy(
            knew_v.at[pl.ds(0, 1), :],
            k_any_out.at[pl.ds(0, 1), :],
            sem).wait()
        return 0

    lax.fori_loop(0, B * A, _wait, 0)


def _run_agent(t, k_flat, a_emb, pos, cur0, upos, uval, tt_row, pvec, p):
    f32 = jnp.float32
    i32 = jnp.int32
    anyspec = pl.BlockSpec(memory_space=pl.ANY)
    vm = pl.BlockSpec(memory_space=pltpu.MemorySpace.VMEM)
    sm = pl.BlockSpec(memory_space=pltpu.MemorySpace.SMEM)
    nhist = 2 if t > 0 else 0
    body = functools.partial(_agent_body, t)
    if t == 0:
        def body(*refs):
            args = refs[:7] + (None, None) + refs[7:]
            return _agent_body(t, *args)
    hist = [upos, uval] if t > 0 else []
    return pl.pallas_call(
        body,
        in_specs=[anyspec, sm] + [vm] * (5 + nhist + 13),
        out_specs=[anyspec, vm, vm, vm, vm],
        out_shape=[
            jax.ShapeDtypeStruct((B * NPAD, D), f32),
            jax.ShapeDtypeStruct((B, A, D), f32),
            jax.ShapeDtypeStruct((B, A, D), f32),
            jax.ShapeDtypeStruct((STEPS, B, A), i32),
            jax.ShapeDtypeStruct((STEPS, B, A, D), f32),
        ],
        input_output_aliases={0: 0},
        scratch_shapes=[
            pltpu.VMEM((B * A, D), f32),
            pltpu.SemaphoreType.DMA,
        ],
    )(k_flat, pos, a_emb, pos, cur0, tt_row, pvec, *hist,
      p['te_W1'], p['te_W2'], p['ag_tW'], p['nd_tW'],
      p['gp_W1'], p['gp_W2'], p['ag_W1'], p['ag_W2'], p['msg_W'],
      p['nd_W1'], p['nd_W2'], p['q_W'], p['k1_W'])


def _argmax_body(t, k_ref, q_ref, pos_ref, vis_ref, cu_ref,
                 pos_out, vis_out, bv_ref, bi_ref):
    j = pl.program_id(0)
    kblk = k_ref[...]
    q = q_ref[...]
    pos = pos_ref[...]

    scores = _bmm(q, kblk, 2) * (1.0 / (D ** 0.5))

    n_g = j * NBK + lax.broadcasted_iota(jnp.int32, (1, NBK), 1)
    member = pos[:, 0][:, None] == n_g
    for a2 in range(1, A):
        member = member | (pos[:, a2][:, None] == n_g)
    if t == 0:
        vis = jnp.where(member, 1.0, 0.0)
    else:
        vis = jnp.where(member, 1.0, vis_ref[...])
    cu = cu_ref[0]
    scores = scores + cu * vis[:, None, :]
    valid = n_g < N
    scores = jnp.where(valid[:, None, :], scores, NEG)
    vis_out[...] = vis * 0.9

    sf = scores.reshape(B * A, NBK)
    bm = jnp.max(sf, axis=1, keepdims=True)
    iot = lax.broadcasted_iota(jnp.int32, (B * A, NBK), 1) + j * NBK
    idx = jnp.min(jnp.where(sf == bm, iot, jnp.int32(2 ** 30)),
                  axis=1, keepdims=True)

    @pl.when(j == 0)
    def _():
        bv_ref[...] = bm
        bi_ref[...] = idx

    @pl.when(j > 0)
    def _():
        upd = bm > bv_ref[...]
        bv_ref[...] = jnp.where(upd, bm, bv_ref[...])
        bi_ref[...] = jnp.where(upd, idx, bi_ref[...])

    @pl.when(j == NB - 1)
    def _():
        pos_out[...] = bi_ref[...].reshape(B, A)


def _run_argmax(t, k3, q, pos, visited, cu):
    f32 = jnp.float32
    i32 = jnp.int32
    body = functools.partial(_argmax_body, t)
    if t == 0:
        def body(k_ref, q_ref, pos_ref, cu_ref, pos_out, vis_out, bv, bi):
            return _argmax_body(t, k_ref, q_ref, pos_ref, None, cu_ref,
                                pos_out, vis_out, bv, bi)
    in_specs = [
        pl.BlockSpec((B, NBK, D), lambda j: (0, j, 0)),
        pl.BlockSpec((B, A, D), lambda j: (0, 0, 0)),
        pl.BlockSpec((B, A), lambda j: (0, 0)),
    ]
    args = [k3, q, pos]
    if t > 0:
        in_specs.append(pl.BlockSpec((B, NBK), lambda j: (0, j)))
        args.append(visited)
    in_specs.append(pl.BlockSpec(memory_space=pltpu.MemorySpace.SMEM))
    args.append(cu)
    return pl.pallas_call(
        body,
        grid=(NB,),
        in_specs=in_specs,
        out_specs=[
            pl.BlockSpec((B, A), lambda j: (0, 0)),
            pl.BlockSpec((B, NBK), lambda j: (0, j)),
        ],
        out_shape=[
            jax.ShapeDtypeStruct((B, A), i32),
            jax.ShapeDtypeStruct((B, NPAD), f32),
        ],
        scratch_shapes=[
            pltpu.VMEM((B * A, 1), f32),
            pltpu.VMEM((B * A, 1), i32),
        ],
    )(*args)


def _fix_body(out_any, upos_sm, upos_ref, uval_ref, oW_ref, pvec,
              out_any_out, rows_v, sem):
    uval = uval_ref[...]
    rows = (_mm(uval.reshape(SA * B, D), oW_ref[...]) + _pv(pvec, 'out_b')
            ).reshape(STEPS, B, A, D)
    rw = rows
    upos4 = upos_ref[...][..., None]
    for jj in range(SA):
        s2, a2 = jj // A, jj % A
        hv = upos4[s2:s2 + 1, :, a2:a2 + 1, :]
        rw = jnp.where(upos4 == hv, rows[s2:s2 + 1, :, a2:a2 + 1, :], rw)
    rows_v[...] = rw.reshape(SA * B, D)

    def _start(i, _):
        a = i % A
        b = (i // A) % B
        s = i // (A * B)
        row = upos_sm[s, b, a] + b * N
        pltpu.make_async_copy(
            rows_v.at[pl.ds(i, 1), :],
            out_any_out.at[pl.ds(row, 1), :],
            sem).start()
        return 0

    lax.fori_loop(0, SA * B, _start, 0)

    def _wait(i, _):
        pltpu.make_async_copy(
            rows_v.at[pl.ds(0, 1), :],
            out_any_out.at[pl.ds(0, 1), :],
            sem).wait()
        return 0

    lax.fori_loop(0, SA * B, _wait, 0)


def _run_fix(out_flat, upos, uval, pvec, p):
    f32 = jnp.float32
    anyspec = pl.BlockSpec(memory_space=pl.ANY)
    vm = pl.BlockSpec(memory_space=pltpu.MemorySpace.VMEM)
    sm = pl.BlockSpec(memory_space=pltpu.MemorySpace.SMEM)
    return pl.pallas_call(
        _fix_body,
        in_specs=[anyspec, sm, vm, vm, vm, vm],
        out_specs=[anyspec],
        out_shape=[jax.ShapeDtypeStruct((B * N, D), f32)],
        input_output_aliases={0: 0},
        scratch_shapes=[
            pltpu.VMEM((SA * B, D), f32),
            pltpu.SemaphoreType.DMA,
        ],
    )(out_flat, upos, upos, uval, p['out_W'], pvec)[0]


def kernel(node_emb, start_pos, params, time_table):
    p = params
    f32 = jnp.float32
    i32 = jnp.int32

    pvec = jnp.concatenate(
        [p[name].astype(f32) for name, _ in _SEG]).reshape(1, PK)
    cu = (p['explored'] - p['unexplored']).reshape(1).astype(f32)

    k3, out_base = _run_k1(node_emb, pvec, p)
    k_flat = k3.reshape(B * NPAD, D)
    out_flat = out_base.reshape(B * N, D)

    sc_gather = _make_sc_gather()
    ne_flat = node_emb.reshape(B * N, D)

    pos = jnp.broadcast_to(start_pos[:, None], (B, A)).astype(i32)
    a_emb = p['agent_emb'].astype(f32)
    upos = None
    uval = None
    visited = None

    for t in range(STEPS):
        cur0 = sc_gather(ne_flat, pos)
        k_flat, a_emb, q, upos, uval = _run_agent(
            t, k_flat, a_emb, pos, cur0, upos, uval, time_table, pvec, p)
        if t < STEPS - 1:
            k3v = k_flat.reshape(B, NPAD, D)
            pos, visited = _run_argmax(t, k3v, q, pos, visited, cu)

    out_flat = _run_fix(out_flat, upos, uval, pvec, p)
    return out_flat.reshape(B, N, D)


# speedup vs baseline: 4.4828x; 1.1357x over previous
import functools

import jax
import jax.numpy as jnp
from jax import lax
from jax.experimental import pallas as pl
from jax.experimental.pallas import tpu as pltpu
from jax.experimental.pallas import tpu_sc as plsc

D = 128
A = 16
STEPS = 4
B = 8
N = 10000
NBK = 1024
NB = 10
NPAD = NB * NBK
SA = STEPS * A
EPS = 1e-5
NEG = -1e30

_SEG = [
    ('te_b1', 256), ('te_b2', 256), ('ag_tb', 384), ('nd_tb', 512),
    ('gp_g', 128), ('gp_b', 128), ('gp_b1', 256), ('gp_b2', 128),
    ('ag_g', 384), ('ag_b', 384), ('ag_b1', 512), ('ag_b2', 256),
    ('msg_g', 128), ('msg_b', 128), ('msg_bb', 128),
    ('nd_g', 512), ('nd_b', 512), ('nd_b1', 512), ('nd_b2', 256),
    ('q_g', 128), ('q_b', 128), ('q_bb', 128),
    ('k1_g', 256), ('k1_b', 256), ('k1_bb', 128), ('out_b', 128),
]
_POFF = {}
_off = 0
for _n, _l in _SEG:
    _POFF[_n] = (_off, _l)
    _off += _l
PK = _off


def _pv(ref, name):
    o, n = _POFF[name]
    return ref[:, o:o + n]


def _lrelu(x):
    return jnp.where(x > 0, x, 0.01 * x)


def _ln(x, g, b):
    m = jnp.mean(x, -1, keepdims=True)
    v = jnp.mean(x * x, -1, keepdims=True) - m * m
    return (x - m) * lax.rsqrt(v + EPS) * g + b


def _mm(x, w):
    return lax.dot_general(x, w, (((x.ndim - 1,), (1,)), ((), ())),
                           preferred_element_type=jnp.float32)


def _bmm(x, y, cdim):
    return lax.dot_general(x, y, (((2,), (cdim,)), ((0,), (0,))),
                           preferred_element_type=jnp.float32)


def _k1_body(ne_ref, pvec, k1W_ref, oW_ref, k_ref, out_ref):
    e = ne_ref[...]
    m = jnp.mean(e, -1, keepdims=True)
    v = jnp.mean(e * e, -1, keepdims=True) - m * m
    nrm = (e - m) * lax.rsqrt(v + EPS)
    k1g = _pv(pvec, 'k1_g')
    k1b = _pv(pvec, 'k1_b')
    gA, gB = k1g[:, :D], k1g[:, D:]
    bA, bB = k1b[:, :D], k1b[:, D:]
    WA = k1W_ref[:, :D]
    WB = k1W_ref[:, D:]
    Wc = WA * gA + WB * gB
    bc = _mm(bA, WA) + _mm(bB, WB) + _pv(pvec, 'k1_bb')
    nf = nrm.reshape(B * NBK, D)
    k_ref[...] = (_mm(nf, Wc) + bc).reshape(B, NBK, D)
    ef = e.reshape(B * NBK, D)
    out_ref[...] = (_mm(ef, oW_ref[...]) + _pv(pvec, 'out_b')
                    ).reshape(B, NBK, D)


def _run_k1(node_emb, pvec, p):
    f32 = jnp.float32
    return pl.pallas_call(
        _k1_body,
        grid=(NB,),
        in_specs=[
            pl.BlockSpec((B, NBK, D), lambda j: (0, j, 0)),
            pl.BlockSpec((1, PK), lambda j: (0, 0)),
            pl.BlockSpec((D, 2 * D), lambda j: (0, 0)),
            pl.BlockSpec((D, D), lambda j: (0, 0)),
        ],
        out_specs=[
            pl.BlockSpec((B, NBK, D), lambda j: (0, j, 0)),
            pl.BlockSpec((B, NBK, D), lambda j: (0, j, 0)),
        ],
        out_shape=[
            jax.ShapeDtypeStruct((B, NPAD, D), f32),
            jax.ShapeDtypeStruct((B, N, D), f32),
        ],
    )(node_emb, pvec, p['k1_W'], p['out_W'])


def _make_sc_gather():
    mesh = plsc.VectorSubcoreMesh(core_axis_name="c", subcore_axis_name="s")
    info = plsc.get_sparse_core_info()
    NC = info.num_cores

    @functools.partial(
        pl.kernel,
        out_type=jax.ShapeDtypeStruct((B * A, D), jnp.float32),
        mesh=mesh,
        scratch_types=[
            pltpu.VMEM((A,), jnp.int32),
            pltpu.VMEM((A, D), jnp.float32),
            pltpu.SemaphoreType.DMA,
        ],
    )
    def gather_rows(tbl_hbm, pos_hbm, out_hbm, idx_v, rows_v, sem):
        wid = lax.axis_index("s") * NC + lax.axis_index("c")

        @pl.when(wid < B)
        def _():
            pltpu.sync_copy(pos_hbm.at[wid], idx_v)
            idx_v[...] = idx_v[...] + wid * N
            pltpu.async_copy(tbl_hbm.at[idx_v], rows_v, sem).wait()
            pltpu.sync_copy(rows_v, out_hbm.at[pl.ds(wid * A, A)])

    return gather_rows


def _agent_body(t, k_any, pos_sm, a_ref, pos_ref, cur0_ref, tt_ref, pvec,
                upos_ref, uval_ref,
                teW1, teW2, agtW, ndtW, gpW1, gpW2, agW1, agW2, msgW,
                ndW1, ndW2, qW, k1W,
                k_any_out, a_out, q_out, upos_out, uval_out,
                knew_v, sem):
    pos = pos_ref[...]
    if t == 0:
        a_emb = jnp.broadcast_to(a_ref[...][None], (B, A, D))
    else:
        a_emb = a_ref[...]
    cur0 = cur0_ref[...].reshape(B, A, D)

    tt = tt_ref[t:t + 1, :]
    t1 = _lrelu(_mm(tt, teW1[...]) + _pv(pvec, 'te_b1'))
    t2 = _mm(t1, teW2[...]) + _pv(pvec, 'te_b2')
    lt2 = _lrelu(t2)
    ag_add = _mm(lt2, agtW[...]) + _pv(pvec, 'ag_tb')
    nd_add = _mm(lt2, ndtW[...]) + _pv(pvec, 'nd_tb')

    pos3 = pos[:, :, None]
    cur = cur0
    for j in range(t * A):
        s, a2 = j // A, j % A
        hv = upos_ref[s, :, a2:a2 + 1][:, :, None]
        cur = jnp.where(pos3 == hv, uval_ref[s, :, a2:a2 + 1, :], cur)

    am = jnp.mean(a_emb, axis=1)
    gv = _ln(am, _pv(pvec, 'gp_g'), _pv(pvec, 'gp_b'))
    gv = _lrelu(_mm(gv, gpW1[...]) + _pv(pvec, 'gp_b1'))
    gvec = _mm(gv, gpW2[...]) + _pv(pvec, 'gp_b2')
    gvec_b = jnp.broadcast_to(gvec[:, None, :], (B, A, D))

    ag_in = jnp.concatenate([a_emb, cur, gvec_b], -1) + ag_add[0]
    h = _ln(ag_in, _pv(pvec, 'ag_g'), _pv(pvec, 'ag_b')).reshape(B * A, 3 * D)
    h = _lrelu(_mm(h, agW1[...]) + _pv(pvec, 'ag_b1'))
    h = _mm(h, agW2[...]) + _pv(pvec, 'ag_b2')
    val, gate = h[:, :D], h[:, D:]
    g = jax.nn.sigmoid(gate)
    a_new = g * a_emb.reshape(B * A, D) + (1 - g) * jnp.tanh(val)
    a_new3 = a_new.reshape(B, A, D)

    msg = _ln(a_new3, _pv(pvec, 'msg_g'), _pv(pvec, 'msg_b'))
    msg = jax.nn.relu(_mm(msg.reshape(B * A, D), msgW[...])
                      + _pv(pvec, 'msg_bb'))
    msg3 = msg.reshape(B, A, D)
    eqm = (pos[:, :, None] == pos[:, None, :]).astype(jnp.float32)
    agg_cur = _bmm(eqm, msg3, 1)

    nd_in = jnp.concatenate([cur, agg_cur, gvec_b, cur0], -1) + nd_add[0]
    h = _ln(nd_in, _pv(pvec, 'nd_g'), _pv(pvec, 'nd_b')).reshape(B * A, 4 * D)
    h = _lrelu(_mm(h, ndW1[...]) + _pv(pvec, 'nd_b1'))
    h = _mm(h, ndW2[...]) + _pv(pvec, 'nd_b2')
    val, gate = h[:, :D], h[:, D:]
    g = jax.nn.sigmoid(gate)
    new_node = g * cur.reshape(B * A, D) + (1 - g) * jnp.tanh(val)
    nn3 = new_node.reshape(B, A, D)

    if t == 0:
        upos_out[...] = jnp.full((STEPS, B, A), -1, jnp.int32)
        uval_out[...] = jnp.zeros((STEPS, B, A, D), jnp.float32)
    else:
        upos_out[...] = upos_ref[...]
        uval_out[...] = uval_ref[...]
    upos_out[t] = pos
    uval_out[t] = nn3

    kin = jnp.concatenate([nn3, cur0], -1)
    kn = _ln(kin, _pv(pvec, 'k1_g'), _pv(pvec, 'k1_b'))
    k_new = (_mm(kn.reshape(B * A, 2 * D), k1W[...]) + _pv(pvec, 'k1_bb')
             ).reshape(B, A, D)
    kw = k_new
    for a2 in range(A):
        eq3 = pos3 == pos3[:, a2:a2 + 1, :]
        kw = jnp.where(eq3, k_new[:, a2:a2 + 1, :], kw)

    a_out[...] = a_new3
    if t == STEPS - 1:
        q_out[...] = a_new3
        return

    q = _ln(a_new, _pv(pvec, 'q_g'), _pv(pvec, 'q_b'))
    q_out[...] = (_mm(q, qW[...]) + _pv(pvec, 'q_bb')).reshape(B, A, D)

    knew_v[...] = kw.reshape(B * A, D)

    def _start(i, _):
        b = i // A
        a = i % A
        row = pos_sm[b, a] + b * NPAD
        pltpu.make_async_copy(
            knew_v.at[pl.ds(i, 1), :],
            k_any_out.at[pl.ds(row, 1), :],
            sem).start()
        return 0

    lax.fori_loop(0, B * A, _start, 0)

    def _wait(i, _):
        pltpu.make_async_copy(
            knew_v.at[pl.ds(0, 1), :],
            k_any_out.at[pl.ds(0, 1), :],
            sem).wait()
        return 0

    lax.fori_loop(0, B * A, _wait, 0)


def _run_agent(t, k_flat, a_emb, pos, cur0, upos, uval, tt_row, pvec, p):
    f32 = jnp.float32
    i32 = jnp.int32
    anyspec = pl.BlockSpec(memory_space=pl.ANY)
    vm = pl.BlockSpec(memory_space=pltpu.MemorySpace.VMEM)
    sm = pl.BlockSpec(memory_space=pltpu.MemorySpace.SMEM)
    nhist = 2 if t > 0 else 0
    body = functools.partial(_agent_body, t)
    if t == 0:
        def body(*refs):
            args = refs[:7] + (None, None) + refs[7:]
            return _agent_body(t, *args)
    hist = [upos, uval] if t > 0 else []
    return pl.pallas_call(
        body,
        in_specs=[anyspec, sm] + [vm] * (5 + nhist + 13),
        out_specs=[anyspec, vm, vm, vm, vm],
        out_shape=[
            jax.ShapeDtypeStruct((B * NPAD, D), f32),
            jax.ShapeDtypeStruct((B, A, D), f32),
            jax.ShapeDtypeStruct((B, A, D), f32),
            jax.ShapeDtypeStruct((STEPS, B, A), i32),
            jax.ShapeDtypeStruct((STEPS, B, A, D), f32),
        ],
        input_output_aliases={0: 0},
        scratch_shapes=[
            pltpu.VMEM((B * A, D), f32),
            pltpu.SemaphoreType.DMA,
        ],
    )(k_flat, pos, a_emb, pos, cur0, tt_row, pvec, *hist,
      p['te_W1'], p['te_W2'], p['ag_tW'], p['nd_tW'],
      p['gp_W1'], p['gp_W2'], p['ag_W1'], p['ag_W2'], p['msg_W'],
      p['nd_W1'], p['nd_W2'], p['q_W'], p['k1_W'])


def _argmax_body(t, k_ref, q_ref, pos_ref, vis_ref, cu_ref,
                 pos_out, vis_out, bv_ref, bi_ref):
    j = pl.program_id(0)
    kblk = k_ref[...]
    q = q_ref[...]
    pos = pos_ref[...]

    scores = _bmm(q, kblk, 2) * (1.0 / (D ** 0.5))

    n_g = j * NBK + lax.broadcasted_iota(jnp.int32, (1, NBK), 1)
    member = pos[:, 0][:, None] == n_g
    for a2 in range(1, A):
        member = member | (pos[:, a2][:, None] == n_g)
    if t == 0:
        vis = jnp.where(member, 1.0, 0.0)
    else:
        vis = jnp.where(member, 1.0, vis_ref[...])
    cu = cu_ref[0]
    scores = scores + cu * vis[:, None, :]
    valid = n_g < N
    scores = jnp.where(valid[:, None, :], scores, NEG)
    vis_out[...] = vis * 0.9

    sf = scores.reshape(B * A, NBK)
    bm = jnp.max(sf, axis=1, keepdims=True)
    iot = lax.broadcasted_iota(jnp.int32, (B * A, NBK), 1) + j * NBK
    idx = jnp.min(jnp.where(sf == bm, iot, jnp.int32(2 ** 30)),
                  axis=1, keepdims=True)

    @pl.when(j == 0)
    def _():
        bv_ref[...] = bm
        bi_ref[...] = idx

    @pl.when(j > 0)
    def _():
        upd = bm > bv_ref[...]
        bv_ref[...] = jnp.where(upd, bm, bv_ref[...])
        bi_ref[...] = jnp.where(upd, idx, bi_ref[...])

    @pl.when(j == NB - 1)
    def _():
        pos_out[...] = bi_ref[...].reshape(B, A)


def _run_argmax(t, k3, q, pos, visited, cu):
    f32 = jnp.float32
    i32 = jnp.int32
    body = functools.partial(_argmax_body, t)
    if t == 0:
        def body(k_ref, q_ref, pos_ref, cu_ref, pos_out, vis_out, bv, bi):
            return _argmax_body(t, k_ref, q_ref, pos_ref, None, cu_ref,
                                pos_out, vis_out, bv, bi)
    in_specs = [
        pl.BlockSpec((B, NBK, D), lambda j: (0, j, 0)),
        pl.BlockSpec((B, A, D), lambda j: (0, 0, 0)),
        pl.BlockSpec((B, A), lambda j: (0, 0)),
    ]
    args = [k3, q, pos]
    if t > 0:
        in_specs.append(pl.BlockSpec((B, NBK), lambda j: (0, j)))
        args.append(visited)
    in_specs.append(pl.BlockSpec(memory_space=pltpu.MemorySpace.SMEM))
    args.append(cu)
    return pl.pallas_call(
        body,
        grid=(NB,),
        in_specs=in_specs,
        out_specs=[
            pl.BlockSpec((B, A), lambda j: (0, 0)),
            pl.BlockSpec((B, NBK), lambda j: (0, j)),
        ],
        out_shape=[
            jax.ShapeDtypeStruct((B, A), i32),
            jax.ShapeDtypeStruct((B, NPAD), f32),
        ],
        scratch_shapes=[
            pltpu.VMEM((B * A, 1), f32),
            pltpu.VMEM((B * A, 1), i32),
        ],
    )(*args)


def _fix_body(out_any, upos_sm, upos_ref, uval_ref, oW_ref, pvec,
              out_any_out, rows_v, sem):
    uval = uval_ref[...]
    rows = (_mm(uval.reshape(SA * B, D), oW_ref[...]) + _pv(pvec, 'out_b')
            ).reshape(STEPS, B, A, D)
    rw = rows
    upos4 = upos_ref[...][..., None]
    for jj in range(SA):
        s2, a2 = jj // A, jj % A
        hv = upos4[s2:s2 + 1, :, a2:a2 + 1, :]
        rw = jnp.where(upos4 == hv, rows[s2:s2 + 1, :, a2:a2 + 1, :], rw)
    rows_v[...] = rw.reshape(SA * B, D)

    def _start(i, _):
        a = i % A
        b = (i // A) % B
        s = i // (A * B)
        row = upos_sm[s, b, a] + b * N
        pltpu.make_async_copy(
            rows_v.at[pl.ds(i, 1), :],
            out_any_out.at[pl.ds(row, 1), :],
            sem).start()
        return 0

    lax.fori_loop(0, SA * B, _start, 0)

    def _wait(i, _):
        pltpu.make_async_copy(
            rows_v.at[pl.ds(0, 1), :],
            out_any_out.at[pl.ds(0, 1), :],
            sem).wait()
        return 0

    lax.fori_loop(0, SA * B, _wait, 0)


def _run_fix(out_flat, upos, uval, pvec, p):
    f32 = jnp.float32
    anyspec = pl.BlockSpec(memory_space=pl.ANY)
    vm = pl.BlockSpec(memory_space=pltpu.MemorySpace.VMEM)
    sm = pl.BlockSpec(memory_space=pltpu.MemorySpace.SMEM)
    return pl.pallas_call(
        _fix_body,
        in_specs=[anyspec, sm, vm, vm, vm, vm],
        out_specs=[anyspec],
        out_shape=[jax.ShapeDtypeStruct((B * N, D), f32)],
        input_output_aliases={0: 0},
        scratch_shapes=[
            pltpu.VMEM((SA * B, D), f32),
            pltpu.SemaphoreType.DMA,
        ],
    )(out_flat, upos, upos, uval, p['out_W'], pvec)[0]


def kernel(node_emb, start_pos, params, time_table):
    p = params
    f32 = jnp.float32
    i32 = jnp.int32

    pvec = jnp.concatenate(
        [p[name].astype(f32) for name, _ in _SEG]).reshape(1, PK)
    cu = (p['explored'] - p['unexplored']).reshape(1).astype(f32)

    k3, out_base = _run_k1(node_emb, pvec, p)
    k_flat = k3.reshape(B * NPAD, D)
    out_flat = out_base.reshape(B * N, D)

    sc_gather = _make_sc_gather()
    ne_flat = node_emb.reshape(B * N, D)

    pos = jnp.broadcast_to(start_pos[:, None], (B, A)).astype(i32)
    a_emb = p['agent_emb'].astype(f32)
    upos = None
    uval = None
    visited = None

    for t in range(STEPS):
        cur0 = sc_gather(ne_flat, pos)
        k_flat, a_emb, q, upos, uval = _run_agent(
            t, k_flat, a_emb, pos, cur0, upos, uval, time_table, pvec, p)
        if t < STEPS - 1:
            k3v = k_flat.reshape(B, NPAD, D)
            pos, visited = _run_argmax(t, k3v, q, pos, visited, cu)

    out_flat = _run_fix(out_flat, upos, uval, pvec, p)
    return out_flat.reshape(B, N, D)


# speedup vs baseline: 4.6548x; 1.0384x over previous
import functools

import jax
import jax.numpy as jnp
from jax import lax
from jax.experimental import pallas as pl
from jax.experimental.pallas import tpu as pltpu
from jax.experimental.pallas import tpu_sc as plsc

D = 128
A = 16
STEPS = 4
B = 8
N = 10000
NBK = 1024
NB = 10
NBKB = 2048
NBB = 5
NPAD = NB * NBK
SA = STEPS * A
EPS = 1e-5
NEG = -1e30

_SEG = [
    ('te_b1', 256), ('te_b2', 256), ('ag_tb', 384), ('nd_tb', 512),
    ('gp_g', 128), ('gp_b', 128), ('gp_b1', 256), ('gp_b2', 128),
    ('ag_g', 384), ('ag_b', 384), ('ag_b1', 512), ('ag_b2', 256),
    ('msg_g', 128), ('msg_b', 128), ('msg_bb', 128),
    ('nd_g', 512), ('nd_b', 512), ('nd_b1', 512), ('nd_b2', 256),
    ('q_g', 128), ('q_b', 128), ('q_bb', 128),
    ('k1_g', 256), ('k1_b', 256), ('k1_bb', 128), ('out_b', 128),
]
_POFF = {}
_off = 0
for _n, _l in _SEG:
    _POFF[_n] = (_off, _l)
    _off += _l
PK = _off


def _pv(ref, name):
    o, n = _POFF[name]
    return ref[:, o:o + n]


def _lrelu(x):
    return jnp.where(x > 0, x, 0.01 * x)


def _ln(x, g, b):
    m = jnp.mean(x, -1, keepdims=True)
    v = jnp.mean(x * x, -1, keepdims=True) - m * m
    return (x - m) * lax.rsqrt(v + EPS) * g + b


def _mm(x, w):
    return lax.dot_general(x, w, (((x.ndim - 1,), (1,)), ((), ())),
                           preferred_element_type=jnp.float32)


def _bmm(x, y, cdim):
    return lax.dot_general(x, y, (((2,), (cdim,)), ((0,), (0,))),
                           preferred_element_type=jnp.float32)


def _k1_body(ne_ref, pvec, k1W_ref, oW_ref, k_ref, out_ref):
    e = ne_ref[...]
    m = jnp.mean(e, -1, keepdims=True)
    v = jnp.mean(e * e, -1, keepdims=True) - m * m
    nrm = (e - m) * lax.rsqrt(v + EPS)
    k1g = _pv(pvec, 'k1_g')
    k1b = _pv(pvec, 'k1_b')
    gA, gB = k1g[:, :D], k1g[:, D:]
    bA, bB = k1b[:, :D], k1b[:, D:]
    WA = k1W_ref[:, :D]
    WB = k1W_ref[:, D:]
    Wc = WA * gA + WB * gB
    bc = _mm(bA, WA) + _mm(bB, WB) + _pv(pvec, 'k1_bb')
    nf = nrm.reshape(B * NBK, D)
    k_ref[...] = (_mm(nf, Wc) + bc).reshape(B, NBK, D)
    ef = e.reshape(B * NBK, D)
    out_ref[...] = (_mm(ef, oW_ref[...]) + _pv(pvec, 'out_b')
                    ).reshape(B, NBK, D)


def _run_k1(node_emb, pvec, p):
    f32 = jnp.float32
    return pl.pallas_call(
        _k1_body,
        grid=(NB,),
        in_specs=[
            pl.BlockSpec((B, NBK, D), lambda j: (0, j, 0)),
            pl.BlockSpec((1, PK), lambda j: (0, 0)),
            pl.BlockSpec((D, 2 * D), lambda j: (0, 0)),
            pl.BlockSpec((D, D), lambda j: (0, 0)),
        ],
        out_specs=[
            pl.BlockSpec((B, NBK, D), lambda j: (0, j, 0)),
            pl.BlockSpec((B, NBK, D), lambda j: (0, j, 0)),
        ],
        out_shape=[
            jax.ShapeDtypeStruct((B, NPAD, D), f32),
            jax.ShapeDtypeStruct((B, N, D), f32),
        ],
    )(node_emb, pvec, p['k1_W'], p['out_W'])


def _make_sc_gather():
    mesh = plsc.VectorSubcoreMesh(core_axis_name="c", subcore_axis_name="s")
    info = plsc.get_sparse_core_info()
    NC = info.num_cores

    @functools.partial(
        pl.kernel,
        out_type=jax.ShapeDtypeStruct((B * A, D), jnp.float32),
        mesh=mesh,
        scratch_types=[
            pltpu.VMEM((A,), jnp.int32),
            pltpu.VMEM((A, D), jnp.float32),
            pltpu.SemaphoreType.DMA,
        ],
    )
    def gather_rows(tbl_hbm, pos_hbm, out_hbm, idx_v, rows_v, sem):
        wid = lax.axis_index("s") * NC + lax.axis_index("c")

        @pl.when(wid < B)
        def _():
            pltpu.sync_copy(pos_hbm.at[wid], idx_v)
            idx_v[...] = idx_v[...] + wid * N
            pltpu.async_copy(tbl_hbm.at[idx_v], rows_v, sem).wait()
            pltpu.sync_copy(rows_v, out_hbm.at[pl.ds(wid * A, A)])

    return gather_rows


def _agent_body(t, k_any, pos_sm, a_ref, pos_ref, cur0_ref, tt_ref, pvec,
                upos_ref, uval_ref,
                teW1, teW2, agtW, ndtW, gpW1, gpW2, agW1, agW2, msgW,
                ndW1, ndW2, qW, k1W,
                k_any_out, a_out, q_out, upos_out, uval_out,
                knew_v, sem):
    pos = pos_ref[...]
    if t == 0:
        a_emb = jnp.broadcast_to(a_ref[...][None], (B, A, D))
    else:
        a_emb = a_ref[...]
    cur0 = cur0_ref[...].reshape(B, A, D)

    tt = tt_ref[t:t + 1, :]
    t1 = _lrelu(_mm(tt, teW1[...]) + _pv(pvec, 'te_b1'))
    t2 = _mm(t1, teW2[...]) + _pv(pvec, 'te_b2')
    lt2 = _lrelu(t2)
    ag_add = _mm(lt2, agtW[...]) + _pv(pvec, 'ag_tb')
    nd_add = _mm(lt2, ndtW[...]) + _pv(pvec, 'nd_tb')

    pos3 = pos[:, :, None]
    cur = cur0
    for j in range(t * A):
        s, a2 = j // A, j % A
        hv = upos_ref[s, :, a2:a2 + 1][:, :, None]
        cur = jnp.where(pos3 == hv, uval_ref[s, :, a2:a2 + 1, :], cur)

    am = jnp.mean(a_emb, axis=1)
    gv = _ln(am, _pv(pvec, 'gp_g'), _pv(pvec, 'gp_b'))
    gv = _lrelu(_mm(gv, gpW1[...]) + _pv(pvec, 'gp_b1'))
    gvec = _mm(gv, gpW2[...]) + _pv(pvec, 'gp_b2')
    gvec_b = jnp.broadcast_to(gvec[:, None, :], (B, A, D))

    ag_in = jnp.concatenate([a_emb, cur, gvec_b], -1) + ag_add[0]
    h = _ln(ag_in, _pv(pvec, 'ag_g'), _pv(pvec, 'ag_b')).reshape(B * A, 3 * D)
    h = _lrelu(_mm(h, agW1[...]) + _pv(pvec, 'ag_b1'))
    h = _mm(h, agW2[...]) + _pv(pvec, 'ag_b2')
    val, gate = h[:, :D], h[:, D:]
    g = jax.nn.sigmoid(gate)
    a_new = g * a_emb.reshape(B * A, D) + (1 - g) * jnp.tanh(val)
    a_new3 = a_new.reshape(B, A, D)

    msg = _ln(a_new3, _pv(pvec, 'msg_g'), _pv(pvec, 'msg_b'))
    msg = jax.nn.relu(_mm(msg.reshape(B * A, D), msgW[...])
                      + _pv(pvec, 'msg_bb'))
    msg3 = msg.reshape(B, A, D)
    eqm = (pos[:, :, None] == pos[:, None, :]).astype(jnp.float32)
    agg_cur = _bmm(eqm, msg3, 1)

    nd_in = jnp.concatenate([cur, agg_cur, gvec_b, cur0], -1) + nd_add[0]
    h = _ln(nd_in, _pv(pvec, 'nd_g'), _pv(pvec, 'nd_b')).reshape(B * A, 4 * D)
    h = _lrelu(_mm(h, ndW1[...]) + _pv(pvec, 'nd_b1'))
    h = _mm(h, ndW2[...]) + _pv(pvec, 'nd_b2')
    val, gate = h[:, :D], h[:, D:]
    g = jax.nn.sigmoid(gate)
    new_node = g * cur.reshape(B * A, D) + (1 - g) * jnp.tanh(val)
    nn3 = new_node.reshape(B, A, D)

    if t == 0:
        upos_out[...] = jnp.full((STEPS, B, A), -1, jnp.int32)
        uval_out[...] = jnp.zeros((STEPS, B, A, D), jnp.float32)
    else:
        upos_out[...] = upos_ref[...]
        uval_out[...] = uval_ref[...]
    upos_out[t] = pos
    uval_out[t] = nn3

    kin = jnp.concatenate([nn3, cur0], -1)
    kn = _ln(kin, _pv(pvec, 'k1_g'), _pv(pvec, 'k1_b'))
    k_new = (_mm(kn.reshape(B * A, 2 * D), k1W[...]) + _pv(pvec, 'k1_bb')
             ).reshape(B, A, D)
    kw = k_new
    for a2 in range(A):
        eq3 = pos3 == pos3[:, a2:a2 + 1, :]
        kw = jnp.where(eq3, k_new[:, a2:a2 + 1, :], kw)

    a_out[...] = a_new3
    if t == STEPS - 1:
        q_out[...] = a_new3
        return

    q = _ln(a_new, _pv(pvec, 'q_g'), _pv(pvec, 'q_b'))
    q_out[...] = (_mm(q, qW[...]) + _pv(pvec, 'q_bb')).reshape(B, A, D)

    knew_v[...] = kw.reshape(B * A, D)

    def _start(i, _):
        b = i // A
        a = i % A
        row = pos_sm[b, a] + b * NPAD
        pltpu.make_async_copy(
            knew_v.at[pl.ds(i, 1), :],
            k_any_out.at[pl.ds(row, 1), :],
            sem).start()
        return 0

    lax.fori_loop(0, B * A, _start, 0)

    def _wait(i, _):
        pltpu.make_async_copy(
            knew_v.at[pl.ds(0, 1), :],
            k_any_out.at[pl.ds(0, 1), :],
            sem).wait()
        return 0

    lax.fori_loop(0, B * A, _wait, 0)


def _run_agent(t, k_flat, a_emb, pos, cur0, upos, uval, tt_row, pvec, p):
    f32 = jnp.float32
    i32 = jnp.int32
    anyspec = pl.BlockSpec(memory_space=pl.ANY)
    vm = pl.BlockSpec(memory_space=pltpu.MemorySpace.VMEM)
    sm = pl.BlockSpec(memory_space=pltpu.MemorySpace.SMEM)
    nhist = 2 if t > 0 else 0
    body = functools.partial(_agent_body, t)
    if t == 0:
        def body(*refs):
            args = refs[:7] + (None, None) + refs[7:]
            return _agent_body(t, *args)
    hist = [upos, uval] if t > 0 else []
    return pl.pallas_call(
        body,
        in_specs=[anyspec, sm] + [vm] * (5 + nhist + 13),
        out_specs=[anyspec, vm, vm, vm, vm],
        out_shape=[
            jax.ShapeDtypeStruct((B * NPAD, D), f32),
            jax.ShapeDtypeStruct((B, A, D), f32),
            jax.ShapeDtypeStruct((B, A, D), f32),
            jax.ShapeDtypeStruct((STEPS, B, A), i32),
            jax.ShapeDtypeStruct((STEPS, B, A, D), f32),
        ],
        input_output_aliases={0: 0},
        scratch_shapes=[
            pltpu.VMEM((B * A, D), f32),
            pltpu.SemaphoreType.DMA,
        ],
    )(k_flat, pos, a_emb, pos, cur0, tt_row, pvec, *hist,
      p['te_W1'], p['te_W2'], p['ag_tW'], p['nd_tW'],
      p['gp_W1'], p['gp_W2'], p['ag_W1'], p['ag_W2'], p['msg_W'],
      p['nd_W1'], p['nd_W2'], p['q_W'], p['k1_W'])


def _argmax_body(t, k_ref, q_ref, pos_ref, vis_ref, cu_ref,
                 pos_out, vis_out, bv_ref, bi_ref):
    j = pl.program_id(0)
    kblk = k_ref[...]
    q = q_ref[...]
    pos = pos_ref[...]

    scores = _bmm(q, kblk, 2) * (1.0 / (D ** 0.5))

    n_g = j * NBKB + lax.broadcasted_iota(jnp.int32, (1, NBKB), 1)
    member = pos[:, 0][:, None] == n_g
    for a2 in range(1, A):
        member = member | (pos[:, a2][:, None] == n_g)
    if t == 0:
        vis = jnp.where(member, 1.0, 0.0)
    else:
        vis = jnp.where(member, 1.0, vis_ref[...])
    cu = cu_ref[0]
    scores = scores + cu * vis[:, None, :]
    valid = n_g < N
    scores = jnp.where(valid[:, None, :], scores, NEG)
    vis_out[...] = vis * 0.9

    sf = scores.reshape(B * A, NBKB)
    bm = jnp.max(sf, axis=1, keepdims=True)
    iot = lax.broadcasted_iota(jnp.int32, (B * A, NBKB), 1) + j * NBKB
    idx = jnp.min(jnp.where(sf == bm, iot, jnp.int32(2 ** 30)),
                  axis=1, keepdims=True)

    @pl.when(j == 0)
    def _():
        bv_ref[...] = bm
        bi_ref[...] = idx

    @pl.when(j > 0)
    def _():
        upd = bm > bv_ref[...]
        bv_ref[...] = jnp.where(upd, bm, bv_ref[...])
        bi_ref[...] = jnp.where(upd, idx, bi_ref[...])

    @pl.when(j == NBB - 1)
    def _():
        pos_out[...] = bi_ref[...].reshape(B, A)


def _run_argmax(t, k3, q, pos, visited, cu):
    f32 = jnp.float32
    i32 = jnp.int32
    body = functools.partial(_argmax_body, t)
    if t == 0:
        def body(k_ref, q_ref, pos_ref, cu_ref, pos_out, vis_out, bv, bi):
            return _argmax_body(t, k_ref, q_ref, pos_ref, None, cu_ref,
                                pos_out, vis_out, bv, bi)
    in_specs = [
        pl.BlockSpec((B, NBKB, D), lambda j: (0, j, 0)),
        pl.BlockSpec((B, A, D), lambda j: (0, 0, 0)),
        pl.BlockSpec((B, A), lambda j: (0, 0)),
    ]
    args = [k3, q, pos]
    if t > 0:
        in_specs.append(pl.BlockSpec((B, NBKB), lambda j: (0, j)))
        args.append(visited)
    in_specs.append(pl.BlockSpec(memory_space=pltpu.MemorySpace.SMEM))
    args.append(cu)
    return pl.pallas_call(
        body,
        grid=(NBB,),
        in_specs=in_specs,
        out_specs=[
            pl.BlockSpec((B, A), lambda j: (0, 0)),
            pl.BlockSpec((B, NBKB), lambda j: (0, j)),
        ],
        out_shape=[
            jax.ShapeDtypeStruct((B, A), i32),
            jax.ShapeDtypeStruct((B, NPAD), f32),
        ],
        scratch_shapes=[
            pltpu.VMEM((B * A, 1), f32),
            pltpu.VMEM((B * A, 1), i32),
        ],
    )(*args)


def _fix_body(out_any, upos_sm, upos_ref, uval_ref, oW_ref, pvec,
              out_any_out, rows_v, sem):
    uval = uval_ref[...]
    rows = (_mm(uval.reshape(SA * B, D), oW_ref[...]) + _pv(pvec, 'out_b')
            ).reshape(STEPS, B, A, D)
    rw = rows
    upos4 = upos_ref[...][..., None]
    for jj in range(SA):
        s2, a2 = jj // A, jj % A
        hv = upos4[s2:s2 + 1, :, a2:a2 + 1, :]
        rw = jnp.where(upos4 == hv, rows[s2:s2 + 1, :, a2:a2 + 1, :], rw)
    rows_v[...] = rw.reshape(SA * B, D)

    def _start(i, _):
        a = i % A
        b = (i // A) % B
        s = i // (A * B)
        row = upos_sm[s, b, a] + b * N
        pltpu.make_async_copy(
            rows_v.at[pl.ds(i, 1), :],
            out_any_out.at[pl.ds(row, 1), :],
            sem).start()
        return 0

    lax.fori_loop(0, SA * B, _start, 0)

    def _wait(i, _):
        pltpu.make_async_copy(
            rows_v.at[pl.ds(0, 1), :],
            out_any_out.at[pl.ds(0, 1), :],
            sem).wait()
        return 0

    lax.fori_loop(0, SA * B, _wait, 0)


def _run_fix(out_flat, upos, uval, pvec, p):
    f32 = jnp.float32
    anyspec = pl.BlockSpec(memory_space=pl.ANY)
    vm = pl.BlockSpec(memory_space=pltpu.MemorySpace.VMEM)
    sm = pl.BlockSpec(memory_space=pltpu.MemorySpace.SMEM)
    return pl.pallas_call(
        _fix_body,
        in_specs=[anyspec, sm, vm, vm, vm, vm],
        out_specs=[anyspec],
        out_shape=[jax.ShapeDtypeStruct((B * N, D), f32)],
        input_output_aliases={0: 0},
        scratch_shapes=[
            pltpu.VMEM((SA * B, D), f32),
            pltpu.SemaphoreType.DMA,
        ],
    )(out_flat, upos, upos, uval, p['out_W'], pvec)[0]


def kernel(node_emb, start_pos, params, time_table):
    p = params
    f32 = jnp.float32
    i32 = jnp.int32

    pvec = jnp.concatenate(
        [p[name].astype(f32) for name, _ in _SEG]).reshape(1, PK)
    cu = (p['explored'] - p['unexplored']).reshape(1).astype(f32)

    k3, out_base = _run_k1(node_emb, pvec, p)
    k_flat = k3.reshape(B * NPAD, D)
    out_flat = out_base.reshape(B * N, D)

    sc_gather = _make_sc_gather()
    ne_flat = node_emb.reshape(B * N, D)

    pos = jnp.broadcast_to(start_pos[:, None], (B, A)).astype(i32)
    a_emb = p['agent_emb'].astype(f32)
    upos = None
    uval = None
    visited = None

    for t in range(STEPS):
        cur0 = sc_gather(ne_flat, pos)
        k_flat, a_emb, q, upos, uval = _run_agent(
            t, k_flat, a_emb, pos, cur0, upos, uval, time_table, pvec, p)
        if t < STEPS - 1:
            k3v = k_flat.reshape(B, NPAD, D)
            pos, visited = _run_argmax(t, k3v, q, pos, visited, cu)

    out_flat = _run_fix(out_flat, upos, uval, pvec, p)
    return out_flat.reshape(B, N, D)


# speedup vs baseline: 4.8949x; 1.0516x over previous
import functools

import jax
import jax.numpy as jnp
from jax import lax
from jax.experimental import pallas as pl
from jax.experimental.pallas import tpu as pltpu
from jax.experimental.pallas import tpu_sc as plsc

D = 128
A = 16
STEPS = 4
B = 8
N = 10000
NBK = 1024
NB = 10
NBKB = 2048
NBB = 5
NPAD = NB * NBK
SA = STEPS * A
EPS = 1e-5
NEG = -1e30

_SEG = [
    ('te_b1', 256), ('te_b2', 256), ('ag_tb', 384), ('nd_tb', 512),
    ('gp_g', 128), ('gp_b', 128), ('gp_b1', 256), ('gp_b2', 128),
    ('ag_g', 384), ('ag_b', 384), ('ag_b1', 512), ('ag_b2', 256),
    ('msg_g', 128), ('msg_b', 128), ('msg_bb', 128),
    ('nd_g', 512), ('nd_b', 512), ('nd_b1', 512), ('nd_b2', 256),
    ('q_g', 128), ('q_b', 128), ('q_bb', 128),
    ('k1_g', 256), ('k1_b', 256), ('k1_bb', 128), ('out_b', 128),
]
_POFF = {}
_off = 0
for _n, _l in _SEG:
    _POFF[_n] = (_off, _l)
    _off += _l
PK = _off


def _pv(ref, name):
    o, n = _POFF[name]
    return ref[:, o:o + n]


def _lrelu(x):
    return jnp.where(x > 0, x, 0.01 * x)


def _ln(x, g, b):
    m = jnp.mean(x, -1, keepdims=True)
    v = jnp.mean(x * x, -1, keepdims=True) - m * m
    return (x - m) * lax.rsqrt(v + EPS) * g + b


def _mm(x, w):
    return lax.dot_general(x, w, (((x.ndim - 1,), (1,)), ((), ())),
                           preferred_element_type=jnp.float32)


def _bmm(x, y, cdim):
    return lax.dot_general(x, y, (((2,), (cdim,)), ((0,), (0,))),
                           preferred_element_type=jnp.float32)


def _k1_body(ne_ref, pvec, k1W_ref, oW_ref, k_ref, out_ref):
    e = ne_ref[...]
    m = jnp.mean(e, -1, keepdims=True)
    v = jnp.mean(e * e, -1, keepdims=True) - m * m
    nrm = (e - m) * lax.rsqrt(v + EPS)
    k1g = _pv(pvec, 'k1_g')
    k1b = _pv(pvec, 'k1_b')
    gA, gB = k1g[:, :D], k1g[:, D:]
    bA, bB = k1b[:, :D], k1b[:, D:]
    WA = k1W_ref[:, :D]
    WB = k1W_ref[:, D:]
    Wc = WA * gA + WB * gB
    bc = _mm(bA, WA) + _mm(bB, WB) + _pv(pvec, 'k1_bb')
    nf = nrm.reshape(B * NBK, D)
    k_ref[...] = (_mm(nf, Wc) + bc).reshape(B, NBK, D)
    ef = e.reshape(B * NBK, D)
    out_ref[...] = (_mm(ef, oW_ref[...]) + _pv(pvec, 'out_b')
                    ).reshape(B, NBK, D)


def _run_k1(node_emb, pvec, p):
    f32 = jnp.float32
    return pl.pallas_call(
        _k1_body,
        grid=(NB,),
        in_specs=[
            pl.BlockSpec((B, NBK, D), lambda j: (0, j, 0)),
            pl.BlockSpec((1, PK), lambda j: (0, 0)),
            pl.BlockSpec((D, 2 * D), lambda j: (0, 0)),
            pl.BlockSpec((D, D), lambda j: (0, 0)),
        ],
        out_specs=[
            pl.BlockSpec((B, NBK, D), lambda j: (0, j, 0)),
            pl.BlockSpec((B, NBK, D), lambda j: (0, j, 0)),
        ],
        out_shape=[
            jax.ShapeDtypeStruct((B, NPAD, D), f32),
            jax.ShapeDtypeStruct((B, N, D), f32),
        ],
    )(node_emb, pvec, p['k1_W'], p['out_W'])


def _make_sc_gather():
    mesh = plsc.VectorSubcoreMesh(core_axis_name="c", subcore_axis_name="s")
    info = plsc.get_sparse_core_info()
    NC = info.num_cores

    @functools.partial(
        pl.kernel,
        out_type=jax.ShapeDtypeStruct((B * A, D), jnp.float32),
        mesh=mesh,
        scratch_types=[
            pltpu.VMEM((A,), jnp.int32),
            pltpu.VMEM((A, D), jnp.float32),
            pltpu.SemaphoreType.DMA,
        ],
    )
    def gather_rows(tbl_hbm, pos_hbm, out_hbm, idx_v, rows_v, sem):
        wid = lax.axis_index("s") * NC + lax.axis_index("c")

        @pl.when(wid < B)
        def _():
            pltpu.sync_copy(pos_hbm.at[wid], idx_v)
            idx_v[...] = idx_v[...] + wid * N
            pltpu.async_copy(tbl_hbm.at[idx_v], rows_v, sem).wait()
            pltpu.sync_copy(rows_v, out_hbm.at[pl.ds(wid * A, A)])

    return gather_rows


def _agent_body(t, k_any, out_any, pos_sm, a_ref, pos_ref, cur0_ref, tt_ref,
                pvec, upos_ref, uval_ref,
                teW1, teW2, agtW, ndtW, gpW1, gpW2, agW1, agW2, msgW,
                ndW1, ndW2, qW, k1W, oW,
                k_any_out, out_any_out, a_out, q_out, upos_out, uval_out,
                knew_v, orow_v, sem):
    pos = pos_ref[...]
    if t == 0:
        a_emb = jnp.broadcast_to(a_ref[...][None], (B, A, D))
    else:
        a_emb = a_ref[...]
    cur0 = cur0_ref[...].reshape(B, A, D)

    tt = tt_ref[t:t + 1, :]
    t1 = _lrelu(_mm(tt, teW1[...]) + _pv(pvec, 'te_b1'))
    t2 = _mm(t1, teW2[...]) + _pv(pvec, 'te_b2')
    lt2 = _lrelu(t2)
    ag_add = _mm(lt2, agtW[...]) + _pv(pvec, 'ag_tb')
    nd_add = _mm(lt2, ndtW[...]) + _pv(pvec, 'nd_tb')

    pos3 = pos[:, :, None]
    cur = cur0
    for j in range(t * A):
        s, a2 = j // A, j % A
        hv = upos_ref[s, :, a2:a2 + 1][:, :, None]
        cur = jnp.where(pos3 == hv, uval_ref[s, :, a2:a2 + 1, :], cur)

    am = jnp.mean(a_emb, axis=1)
    gv = _ln(am, _pv(pvec, 'gp_g'), _pv(pvec, 'gp_b'))
    gv = _lrelu(_mm(gv, gpW1[...]) + _pv(pvec, 'gp_b1'))
    gvec = _mm(gv, gpW2[...]) + _pv(pvec, 'gp_b2')
    gvec_b = jnp.broadcast_to(gvec[:, None, :], (B, A, D))

    ag_in = jnp.concatenate([a_emb, cur, gvec_b], -1) + ag_add[0]
    h = _ln(ag_in, _pv(pvec, 'ag_g'), _pv(pvec, 'ag_b')).reshape(B * A, 3 * D)
    h = _lrelu(_mm(h, agW1[...]) + _pv(pvec, 'ag_b1'))
    h = _mm(h, agW2[...]) + _pv(pvec, 'ag_b2')
    val, gate = h[:, :D], h[:, D:]
    g = jax.nn.sigmoid(gate)
    a_new = g * a_emb.reshape(B * A, D) + (1 - g) * jnp.tanh(val)
    a_new3 = a_new.reshape(B, A, D)

    msg = _ln(a_new3, _pv(pvec, 'msg_g'), _pv(pvec, 'msg_b'))
    msg = jax.nn.relu(_mm(msg.reshape(B * A, D), msgW[...])
                      + _pv(pvec, 'msg_bb'))
    msg3 = msg.reshape(B, A, D)
    eqm = (pos[:, :, None] == pos[:, None, :]).astype(jnp.float32)
    agg_cur = _bmm(eqm, msg3, 1)

    nd_in = jnp.concatenate([cur, agg_cur, gvec_b, cur0], -1) + nd_add[0]
    h = _ln(nd_in, _pv(pvec, 'nd_g'), _pv(pvec, 'nd_b')).reshape(B * A, 4 * D)
    h = _lrelu(_mm(h, ndW1[...]) + _pv(pvec, 'nd_b1'))
    h = _mm(h, ndW2[...]) + _pv(pvec, 'nd_b2')
    val, gate = h[:, :D], h[:, D:]
    g = jax.nn.sigmoid(gate)
    new_node = g * cur.reshape(B * A, D) + (1 - g) * jnp.tanh(val)
    nn3 = new_node.reshape(B, A, D)

    if t == 0:
        upos_out[...] = jnp.full((STEPS, B, A), -1, jnp.int32)
        uval_out[...] = jnp.zeros((STEPS, B, A, D), jnp.float32)
    else:
        upos_out[...] = upos_ref[...]
        uval_out[...] = uval_ref[...]
    upos_out[t] = pos
    uval_out[t] = nn3

    kin = jnp.concatenate([nn3, cur0], -1)
    kn = _ln(kin, _pv(pvec, 'k1_g'), _pv(pvec, 'k1_b'))
    k_new = (_mm(kn.reshape(B * A, 2 * D), k1W[...]) + _pv(pvec, 'k1_bb')
             ).reshape(B, A, D)
    kw = k_new
    for a2 in range(A):
        eq3 = pos3 == pos3[:, a2:a2 + 1, :]
        kw = jnp.where(eq3, k_new[:, a2:a2 + 1, :], kw)

    a_out[...] = a_new3
    if t == STEPS - 1:
        q_out[...] = a_new3
    else:
        q = _ln(a_new, _pv(pvec, 'q_g'), _pv(pvec, 'q_b'))
        q_out[...] = (_mm(q, qW[...]) + _pv(pvec, 'q_bb')).reshape(B, A, D)
        knew_v[...] = kw.reshape(B * A, D)

    nnw = nn3
    for a2 in range(A):
        eq3 = pos3 == pos3[:, a2:a2 + 1, :]
        nnw = jnp.where(eq3, nn3[:, a2:a2 + 1, :], nnw)
    orow_v[...] = _mm(nnw.reshape(B * A, D), oW[...]) + _pv(pvec, 'out_b')

    def _start_o(i, _):
        b = i // A
        a = i % A
        row = pos_sm[b, a] + b * N
        pltpu.make_async_copy(
            orow_v.at[pl.ds(i, 1), :],
            out_any_out.at[pl.ds(row, 1), :],
            sem).start()
        return 0

    lax.fori_loop(0, B * A, _start_o, 0)
    ncopies = B * A

    if t < STEPS - 1:
        def _start_k(i, _):
            b = i // A
            a = i % A
            row = pos_sm[b, a] + b * NPAD
            pltpu.make_async_copy(
                knew_v.at[pl.ds(i, 1), :],
                k_any_out.at[pl.ds(row, 1), :],
                sem).start()
            return 0

        lax.fori_loop(0, B * A, _start_k, 0)
        ncopies += B * A

    def _wait(i, _):
        pltpu.make_async_copy(
            orow_v.at[pl.ds(0, 1), :],
            out_any_out.at[pl.ds(0, 1), :],
            sem).wait()
        return 0

    lax.fori_loop(0, ncopies, _wait, 0)


def _run_agent(t, k_flat, out_flat, a_emb, pos, cur0, upos, uval, tt_row,
               pvec, p):
    f32 = jnp.float32
    i32 = jnp.int32
    anyspec = pl.BlockSpec(memory_space=pl.ANY)
    vm = pl.BlockSpec(memory_space=pltpu.MemorySpace.VMEM)
    sm = pl.BlockSpec(memory_space=pltpu.MemorySpace.SMEM)
    nhist = 2 if t > 0 else 0
    body = functools.partial(_agent_body, t)
    if t == 0:
        def body(*refs):
            args = refs[:8] + (None, None) + refs[8:]
            return _agent_body(t, *args)
    hist = [upos, uval] if t > 0 else []
    return pl.pallas_call(
        body,
        in_specs=[anyspec, anyspec, sm] + [vm] * (5 + nhist + 14),
        out_specs=[anyspec, anyspec, vm, vm, vm, vm],
        out_shape=[
            jax.ShapeDtypeStruct((B * NPAD, D), f32),
            jax.ShapeDtypeStruct((B * N, D), f32),
            jax.ShapeDtypeStruct((B, A, D), f32),
            jax.ShapeDtypeStruct((B, A, D), f32),
            jax.ShapeDtypeStruct((STEPS, B, A), i32),
            jax.ShapeDtypeStruct((STEPS, B, A, D), f32),
        ],
        input_output_aliases={0: 0, 1: 1},
        scratch_shapes=[
            pltpu.VMEM((B * A, D), f32),
            pltpu.VMEM((B * A, D), f32),
            pltpu.SemaphoreType.DMA,
        ],
    )(k_flat, out_flat, pos, a_emb, pos, cur0, tt_row, pvec, *hist,
      p['te_W1'], p['te_W2'], p['ag_tW'], p['nd_tW'],
      p['gp_W1'], p['gp_W2'], p['ag_W1'], p['ag_W2'], p['msg_W'],
      p['nd_W1'], p['nd_W2'], p['q_W'], p['k1_W'], p['out_W'])


def _argmax_body(t, k_ref, q_ref, pos_ref, vis_ref, cu_ref,
                 pos_out, vis_out, bv_ref, bi_ref):
    j = pl.program_id(0)
    kblk = k_ref[...]
    q = q_ref[...]
    pos = pos_ref[...]

    scores = _bmm(q, kblk, 2) * (1.0 / (D ** 0.5))

    n_g = j * NBKB + lax.broadcasted_iota(jnp.int32, (1, NBKB), 1)
    member = pos[:, 0][:, None] == n_g
    for a2 in range(1, A):
        member = member | (pos[:, a2][:, None] == n_g)
    if t == 0:
        vis = jnp.where(member, 1.0, 0.0)
    else:
        vis = jnp.where(member, 1.0, vis_ref[...])
    cu = cu_ref[0]
    scores = scores + cu * vis[:, None, :]
    valid = n_g < N
    scores = jnp.where(valid[:, None, :], scores, NEG)
    vis_out[...] = vis * 0.9

    sf = scores.reshape(B * A, NBKB)
    bm = jnp.max(sf, axis=1, keepdims=True)
    iot = lax.broadcasted_iota(jnp.int32, (B * A, NBKB), 1) + j * NBKB
    idx = jnp.min(jnp.where(sf == bm, iot, jnp.int32(2 ** 30)),
                  axis=1, keepdims=True)

    @pl.when(j == 0)
    def _():
        bv_ref[...] = bm
        bi_ref[...] = idx

    @pl.when(j > 0)
    def _():
        upd = bm > bv_ref[...]
        bv_ref[...] = jnp.where(upd, bm, bv_ref[...])
        bi_ref[...] = jnp.where(upd, idx, bi_ref[...])

    @pl.when(j == NBB - 1)
    def _():
        pos_out[...] = bi_ref[...].reshape(B, A)


def _run_argmax(t, k3, q, pos, visited, cu):
    f32 = jnp.float32
    i32 = jnp.int32
    body = functools.partial(_argmax_body, t)
    if t == 0:
        def body(k_ref, q_ref, pos_ref, cu_ref, pos_out, vis_out, bv, bi):
            return _argmax_body(t, k_ref, q_ref, pos_ref, None, cu_ref,
                                pos_out, vis_out, bv, bi)
    in_specs = [
        pl.BlockSpec((B, NBKB, D), lambda j: (0, j, 0)),
        pl.BlockSpec((B, A, D), lambda j: (0, 0, 0)),
        pl.BlockSpec((B, A), lambda j: (0, 0)),
    ]
    args = [k3, q, pos]
    if t > 0:
        in_specs.append(pl.BlockSpec((B, NBKB), lambda j: (0, j)))
        args.append(visited)
    in_specs.append(pl.BlockSpec(memory_space=pltpu.MemorySpace.SMEM))
    args.append(cu)
    return pl.pallas_call(
        body,
        grid=(NBB,),
        in_specs=in_specs,
        out_specs=[
            pl.BlockSpec((B, A), lambda j: (0, 0)),
            pl.BlockSpec((B, NBKB), lambda j: (0, j)),
        ],
        out_shape=[
            jax.ShapeDtypeStruct((B, A), i32),
            jax.ShapeDtypeStruct((B, NPAD), f32),
        ],
        scratch_shapes=[
            pltpu.VMEM((B * A, 1), f32),
            pltpu.VMEM((B * A, 1), i32),
        ],
    )(*args)


def kernel(node_emb, start_pos, params, time_table):
    p = params
    f32 = jnp.float32
    i32 = jnp.int32

    pvec = jnp.concatenate(
        [p[name].astype(f32) for name, _ in _SEG]).reshape(1, PK)
    cu = (p['explored'] - p['unexplored']).reshape(1).astype(f32)

    k3, out_base = _run_k1(node_emb, pvec, p)
    k_flat = k3.reshape(B * NPAD, D)
    out_flat = out_base.reshape(B * N, D)

    sc_gather = _make_sc_gather()
    ne_flat = node_emb.reshape(B * N, D)

    pos = jnp.broadcast_to(start_pos[:, None], (B, A)).astype(i32)
    a_emb = p['agent_emb'].astype(f32)
    upos = None
    uval = None
    visited = None

    for t in range(STEPS):
        cur0 = sc_gather(ne_flat, pos)
        k_flat, out_flat, a_emb, q, upos, uval = _run_agent(
            t, k_flat, out_flat, a_emb, pos, cur0, upos, uval, time_table,
            pvec, p)
        if t < STEPS - 1:
            k3v = k_flat.reshape(B, NPAD, D)
            pos, visited = _run_argmax(t, k3v, q, pos, visited, cu)

    return out_flat.reshape(B, N, D)
